# baseline (speedup 1.0000x reference)
import functools

import jax
import jax.numpy as jnp
from jax import lax
from jax.experimental import pallas as pl
from jax.experimental.pallas import tpu as pltpu
from jax.experimental.pallas import tpu_sc as plsc

N = 50000
D = 64
E = 800000
S2 = 16

NT = 16
NC = 2
N1 = 50432
RPT = N1 // NT
ZCH = 200
ZTAIL = RPT - (RPT // ZCH) * ZCH
CHUNK = 128
E_PAD = 802816
EPT = E_PAD // NT
NCH = EPT // CHUNK
E_PAD_D = 819200
NCH_D = E_PAD_D // (NT * NC * CHUNK)

RNG = 16896
NRNG = 3
RPT_A = RNG // NT
ZT_A = RPT_A - (RPT_A // ZCH) * ZCH
SL = 52008

BN = 1576
NB = N1 // BN
GB = SL // BN

_mesh = plsc.VectorSubcoreMesh(core_axis_name="c", subcore_axis_name="s")


def _zero_rows(zbuf, acc, r0, rows):
    for k in range(rows // ZCH):
        pltpu.sync_copy(zbuf, acc.at[pl.ds(r0 + k * ZCH, ZCH), :])
    tail = rows - (rows // ZCH) * ZCH
    if tail:
        pltpu.sync_copy(zbuf.at[pl.ds(0, tail), :],
                        acc.at[pl.ds(r0 + (rows // ZCH) * ZCH, tail), :])


@functools.partial(
    pl.kernel,
    out_type=jax.ShapeDtypeStruct((2 * N1, 1), jnp.float32),
    mesh=_mesh,
    scratch_types=[
        pltpu.VMEM((NCH_D, CHUNK), jnp.int32),
        pltpu.VMEM((CHUNK, 1), jnp.float32),
        pltpu.VMEM((ZCH, 1), jnp.float32),
        pltpu.VMEM_SHARED((N1, 1), jnp.float32),
    ],
    compiler_params=pltpu.CompilerParams(use_tc_tiling_on_sc=False),
)
def _deg_kernel(dst2d_hbm, ones_hbm, zeros_hbm, deg_hbm,
                dst_v, ones_v, zbuf, acc):
    c = lax.axis_index("c")
    s = lax.axis_index("s")
    r0 = s * RPT
    pltpu.sync_copy(ones_hbm, ones_v)
    pltpu.sync_copy(zeros_hbm, zbuf)
    pltpu.sync_copy(dst2d_hbm.at[pl.ds((c * NT + s) * NCH_D, NCH_D), :], dst_v)
    _zero_rows(zbuf, acc, r0, RPT)
    plsc.subcore_barrier()

    def body(j, carry):
        pltpu.sync_copy(ones_v, acc.at[dst_v.at[j]], add=True)
        return carry

    lax.fori_loop(0, NCH_D, body, 0)
    plsc.subcore_barrier()
    pltpu.sync_copy(acc.at[pl.ds(r0, RPT), :],
                    deg_hbm.at[pl.ds(c * N1 + r0, RPT), :])


@functools.partial(
    pl.kernel,
    out_type=jax.ShapeDtypeStruct((4 * SL, 16), jnp.float32),
    mesh=_mesh,
    scratch_types=[
        pltpu.VMEM((NCH, CHUNK), jnp.int32),
        pltpu.VMEM((NCH, CHUNK), jnp.int32),
        pltpu.VMEM((CHUNK, 16), jnp.float32),
        pltpu.VMEM((CHUNK, 16), jnp.float32),
        pltpu.VMEM((ZCH, 16), jnp.float32),
        pltpu.VMEM_SHARED((RNG + 2 * NT, 16), jnp.float32),
        pltpu.SemaphoreType.DMA,
        pltpu.SemaphoreType.DMA,
    ],
    compiler_params=pltpu.CompilerParams(use_tc_tiling_on_sc=False),
)
def _spmm_kernel(ys_hbm, src4_hbm, dst0_hbm, dst1_hbm, dst2_hbm, zeros_hbm,
                 out_hbm, src_v, dst_v, rows0, rows1, zbuf, acc, sem0, sem1):
    c = lax.axis_index("c")
    s = lax.axis_index("s")
    r0 = s * RPT_A
    dsts = (dst0_hbm, dst1_hbm, dst2_hbm)
    pltpu.sync_copy(zeros_hbm, zbuf)

    def gather(j, buf, sem):
        pltpu.async_copy(ys_hbm.at[src_v.at[j]], buf, sem)

    def wait(buf, sem):
        pltpu.make_async_copy(ys_hbm.at[pl.ds(0, CHUNK), :], buf, sem).wait()

    for p in range(2):
        g = 2 * c + p
        for r in range(NRNG):
            pltpu.sync_copy(
                src4_hbm.at[pl.ds((g * NRNG + r) * (E_PAD // CHUNK)
                                  + s * NCH, NCH), :], src_v)
            pltpu.sync_copy(dsts[r].at[pl.ds(s * NCH, NCH), :], dst_v)
            _zero_rows(zbuf, acc, r0, RPT_A)

            plsc.subcore_barrier()

            gather(0, rows0, sem0)
            gather(1, rows1, sem1)

            def body(i2, carry):
                j = i2 * 2
                wait(rows0, sem0)

                @pl.when(i2 < NCH // 2 - 1)
                def _():
                    gather(j + 2, rows0, sem0)

                pltpu.sync_copy(rows0, acc.at[dst_v.at[j]], add=True)
                wait(rows1, sem1)

                @pl.when(i2 < NCH // 2 - 1)
                def _():
                    gather(j + 3, rows1, sem1)

                pltpu.sync_copy(rows1, acc.at[dst_v.at[j + 1]], add=True)
                return carry

            lax.fori_loop(0, NCH // 2, body, 0)
            plsc.subcore_barrier()

            pltpu.sync_copy(acc.at[pl.ds(r0, RPT_A), :],
                            out_hbm.at[pl.ds(g * SL + r * RNG + r0, RPT_A), :])
            plsc.subcore_barrier()


def _mm_body(x_ref, w_ref, o_ref):
    o_ref[...] = jnp.dot(x_ref[...], w_ref[...], preferred_element_type=jnp.float32)


def _scale_body(xw_ref, d0_ref, d1_ref, ys_ref, dis_ref):
    deg = d0_ref[...] + d1_ref[...] + 1.0
    dis = lax.rsqrt(deg)
    dis_ref[...] = dis
    ys_ref[...] = dis * xw_ref[...]


def _mid_body(a0_ref, a1_ref, a2_ref, a3_ref, xw_ref, x_ref, dis_ref, w_ref,
              b_ref, xw2_ref, ys2_ref):
    acc = jnp.concatenate(
        [a0_ref[...], a1_ref[...], a2_ref[...], a3_ref[...]], axis=1)
    d = dis_ref[...]
    h = jnp.maximum(d * acc + (d * d) * xw_ref[...] + b_ref[...] + x_ref[...], 0.0)
    xw2 = jnp.dot(h, w_ref[...], preferred_element_type=jnp.float32)
    xw2_ref[...] = xw2
    ys2_ref[...] = d * xw2


def _head_body(a0_ref, a1_ref, a2_ref, a3_ref, xw_ref, x_ref, dis_ref, wl_ref,
               b_ref, bl_ref, s_ref):
    acc = jnp.concatenate(
        [a0_ref[...], a1_ref[...], a2_ref[...], a3_ref[...]], axis=1)
    d = dis_ref[...]
    h = jnp.maximum(d * acc + (d * d) * xw_ref[...] + b_ref[...] + x_ref[...], 0.0)
    s_ref[...] = jnp.dot(h, wl_ref[...], preferred_element_type=jnp.float32) + bl_ref[...]


def _row_spec(w):
    return pl.BlockSpec((BN, w), lambda i: (i, 0))


def _off_spec(w, g, stride):
    return pl.BlockSpec((BN, w), lambda i, g=g, stride=stride: (g * stride + i, 0))


def _full_spec(shape):
    return pl.BlockSpec(shape, lambda i: tuple(0 for _ in shape))


def _f32(shape):
    return jax.ShapeDtypeStruct(shape, jnp.float32)


def kernel(x, edge_index, batch, n_cells, W2, b2, W3, b3, Wl, bl):
    del batch, n_cells
    src = edge_index[0]
    dst = edge_index[1]
    pad_i = jnp.full((E_PAD - E,), N1 - 1, dtype=jnp.int32)
    src_p = jnp.concatenate([src, pad_i])
    dst_p = jnp.concatenate([dst, pad_i])
    src4 = jnp.concatenate(
        [jnp.where((dst_p >= r * RNG) & (dst_p < (r + 1) * RNG),
                   4 * src_p + g, 4 * (N1 - 1) + g)
         for g in range(4) for r in range(NRNG)]
    ).reshape(4 * NRNG * (E_PAD // CHUNK), CHUNK)
    eidx = jnp.arange(E_PAD, dtype=jnp.int32)
    dump = RNG + 2 * (eidx // EPT) + (eidx // CHUNK) % 2
    dst3 = []
    for r in range(NRNG):
        base = r * RNG
        loc = jnp.where((dst_p >= base) & (dst_p < base + RNG),
                        dst_p - base, dump).astype(jnp.int32)
        dst3.append(loc.reshape(E_PAD // CHUNK, CHUNK))
    dst2d_deg = jnp.concatenate(
        [dst, jnp.full((E_PAD_D - E,), N1 - 1, dtype=jnp.int32)]
    ).reshape(E_PAD_D // CHUNK, CHUNK)
    x_p = jnp.concatenate([x, jnp.zeros((N1 - N, D), jnp.float32)])

    ones_h = jnp.ones((CHUNK, 1), jnp.float32)
    zeros1_h = jnp.zeros((ZCH, 1), jnp.float32)
    zeros16_h = jnp.zeros((ZCH, 16), jnp.float32)
    b2r = b2.reshape(1, D)
    b3r = b3.reshape(1, D)
    blr = bl.reshape(1, S2)

    degs = _deg_kernel(dst2d_deg, ones_h, zeros1_h)

    xw1 = pl.pallas_call(
        _mm_body,
        grid=(NB,),
        in_specs=[_row_spec(D), _full_spec((D, D))],
        out_specs=_row_spec(D),
        out_shape=_f32((N1, D)),
    )(x_p, W2)

    ys1, dis = pl.pallas_call(
        _scale_body,
        grid=(NB,),
        in_specs=[_row_spec(D), _off_spec(1, 0, NB), _off_spec(1, 1, NB)],
        out_specs=[_row_spec(D), _row_spec(1)],
        out_shape=[_f32((N1, D)), _f32((N1, 1))],
    )(xw1, degs, degs)

    acc1 = _spmm_kernel(ys1.reshape(4 * N1, 16), src4,
                        dst3[0], dst3[1], dst3[2], zeros16_h)

    xw2, ys2 = pl.pallas_call(
        _mid_body,
        grid=(NB,),
        in_specs=[_off_spec(16, g, GB) for g in range(4)] + [
            _row_spec(D), _row_spec(D),
            _row_spec(1), _full_spec((D, D)), _full_spec((1, D))],
        out_specs=[_row_spec(D), _row_spec(D)],
        out_shape=[_f32((N1, D)), _f32((N1, D))],
    )(acc1, acc1, acc1, acc1, xw1, x_p, dis, W3, b2r)

    acc2 = _spmm_kernel(ys2.reshape(4 * N1, 16), src4,
                        dst3[0], dst3[1], dst3[2], zeros16_h)

    s_full = pl.pallas_call(
        _head_body,
        grid=(NB,),
        in_specs=[_off_spec(16, g, GB) for g in range(4)] + [
            _row_spec(D), _row_spec(D),
            _row_spec(1), _full_spec((D, S2)), _full_spec((1, D)),
            _full_spec((1, S2))],
        out_specs=_row_spec(S2),
        out_shape=_f32((N1, S2)),
    )(acc2, acc2, acc2, acc2, xw2, x_p, dis, Wl, b3r, blr)

    return s_full[:N]


# speedup vs baseline: 32.1622x; 32.1622x over previous
import functools

import jax
import jax.numpy as jnp
from jax import lax
from jax.experimental import pallas as pl
from jax.experimental.pallas import tpu as pltpu
from jax.experimental.pallas import tpu_sc as plsc

N = 50000
D = 64
E = 800000
S2 = 16

NT = 16
NC = 2
N1 = 50432
RPT = N1 // NT
ZCH = 200
ZTAIL = RPT - (RPT // ZCH) * ZCH
CHUNK = 128
E_PAD = 802816
EPT = E_PAD // NT
NCH = EPT // CHUNK
E_PAD_D = 819200
NCH_D = E_PAD_D // (NT * NC * CHUNK)

RNG = 16896
NRNG = 3
RPT_A = RNG // NT
ZT_A = RPT_A - (RPT_A // ZCH) * ZCH
SL = 52008

BN = 1576
NB = N1 // BN
GB = SL // BN

_mesh = plsc.VectorSubcoreMesh(core_axis_name="c", subcore_axis_name="s")


def _zero_rows(zbuf, acc, r0, rows):
    for k in range(rows // ZCH):
        pltpu.sync_copy(zbuf, acc.at[pl.ds(r0 + k * ZCH, ZCH), :])
    tail = rows - (rows // ZCH) * ZCH
    if tail:
        pltpu.sync_copy(zbuf.at[pl.ds(0, tail), :],
                        acc.at[pl.ds(r0 + (rows // ZCH) * ZCH, tail), :])


@functools.partial(
    pl.kernel,
    out_type=jax.ShapeDtypeStruct((2 * N1, 1), jnp.float32),
    mesh=_mesh,
    scratch_types=[
        pltpu.VMEM((NCH_D, CHUNK), jnp.int32),
        pltpu.VMEM((CHUNK, 1), jnp.float32),
        pltpu.VMEM((ZCH, 1), jnp.float32),
        pltpu.VMEM_SHARED((N1, 1), jnp.float32),
    ],
    compiler_params=pltpu.CompilerParams(use_tc_tiling_on_sc=False),
)
def _deg_kernel(dst2d_hbm, ones_hbm, zeros_hbm, deg_hbm,
                dst_v, ones_v, zbuf, acc):
    c = lax.axis_index("c")
    s = lax.axis_index("s")
    r0 = s * RPT
    pltpu.sync_copy(ones_hbm, ones_v)
    pltpu.sync_copy(zeros_hbm, zbuf)
    pltpu.sync_copy(dst2d_hbm.at[pl.ds((c * NT + s) * NCH_D, NCH_D), :], dst_v)
    _zero_rows(zbuf, acc, r0, RPT)
    plsc.subcore_barrier()

    def body(j, carry):
        pltpu.sync_copy(ones_v, acc.at[dst_v.at[j]], add=True)
        return carry

    lax.fori_loop(0, NCH_D, body, 0)
    plsc.subcore_barrier()
    pltpu.sync_copy(acc.at[pl.ds(r0, RPT), :],
                    deg_hbm.at[pl.ds(c * N1 + r0, RPT), :])


@functools.partial(
    pl.kernel,
    out_type=jax.ShapeDtypeStruct((4 * SL, 16), jnp.float32),
    mesh=_mesh,
    scratch_types=[
        pltpu.VMEM((NCH, CHUNK), jnp.int32),
        pltpu.VMEM((NCH, CHUNK), jnp.int32),
        pltpu.VMEM((CHUNK, 16), jnp.float32),
        pltpu.VMEM((CHUNK, 16), jnp.float32),
        pltpu.VMEM((CHUNK, 16), jnp.float32),
        pltpu.VMEM((CHUNK, 16), jnp.float32),
        pltpu.VMEM((ZCH, 16), jnp.float32),
        pltpu.VMEM_SHARED((RNG + 2 * NT, 16), jnp.float32),
        pltpu.SemaphoreType.DMA,
        pltpu.SemaphoreType.DMA,
        pltpu.SemaphoreType.DMA,
        pltpu.SemaphoreType.DMA,
    ],
    compiler_params=pltpu.CompilerParams(use_tc_tiling_on_sc=False),
)
def _spmm_kernel(ys_hbm, src4_hbm, dst0_hbm, dst1_hbm, dst2_hbm, zeros_hbm,
                 out_hbm, src_v, dst_v, rows0, rows1, rows2, rows3, zbuf,
                 acc, sem0, sem1, sem2, sem3):
    c = lax.axis_index("c")
    s = lax.axis_index("s")
    r0 = s * RPT_A
    dsts = (dst0_hbm, dst1_hbm, dst2_hbm)
    pltpu.sync_copy(zeros_hbm, zbuf)

    def gather(j, buf, sem):
        pltpu.async_copy(ys_hbm.at[src_v.at[j]], buf, sem)

    def wait(buf, sem):
        pltpu.make_async_copy(ys_hbm.at[pl.ds(0, CHUNK), :], buf, sem).wait()

    for p in range(2):
        g = 2 * c + p
        pltpu.sync_copy(
            src4_hbm.at[pl.ds(g * (E_PAD // CHUNK) + s * NCH, NCH), :], src_v)
        for r in range(NRNG):
            pltpu.sync_copy(dsts[r].at[pl.ds(s * NCH, NCH), :], dst_v)
            _zero_rows(zbuf, acc, r0, RPT_A)

            plsc.subcore_barrier()

            ring = ((rows0, sem0), (rows1, sem1), (rows2, sem2), (rows3, sem3))
            for b, (buf, sem) in enumerate(ring):
                gather(b, buf, sem)

            def body(i4, carry):
                j = i4 * 4
                for b, (buf, sem) in enumerate(ring):
                    wait(buf, sem)

                    @pl.when(i4 < NCH // 4 - 1)
                    def _():
                        gather(j + 4 + b, buf, sem)

                    pltpu.sync_copy(buf, acc.at[dst_v.at[j + b]], add=True)
                return carry

            lax.fori_loop(0, NCH // 4, body, 0)
            plsc.subcore_barrier()

            pltpu.sync_copy(acc.at[pl.ds(r0, RPT_A), :],
                            out_hbm.at[pl.ds(g * SL + r * RNG + r0, RPT_A), :])
            plsc.subcore_barrier()


def _mm_body(x_ref, w_ref, o_ref):
    o_ref[...] = jnp.dot(x_ref[...], w_ref[...], preferred_element_type=jnp.float32)


def _scale_body(xw_ref, d0_ref, d1_ref, ys_ref, dis_ref):
    deg = d0_ref[...] + d1_ref[...] + 1.0
    dis = lax.rsqrt(deg)
    dis_ref[...] = dis
    ys_ref[...] = dis * xw_ref[...]


def _mid_body(a0_ref, a1_ref, a2_ref, a3_ref, xw_ref, x_ref, dis_ref, w_ref,
              b_ref, xw2_ref, ys2_ref):
    acc = jnp.concatenate(
        [a0_ref[...], a1_ref[...], a2_ref[...], a3_ref[...]], axis=1)
    d = dis_ref[...]
    h = jnp.maximum(d * acc + (d * d) * xw_ref[...] + b_ref[...] + x_ref[...], 0.0)
    xw2 = jnp.dot(h, w_ref[...], preferred_element_type=jnp.float32)
    xw2_ref[...] = xw2
    ys2_ref[...] = d * xw2


def _head_body(a0_ref, a1_ref, a2_ref, a3_ref, xw_ref, x_ref, dis_ref, wl_ref,
               b_ref, bl_ref, s_ref):
    acc = jnp.concatenate(
        [a0_ref[...], a1_ref[...], a2_ref[...], a3_ref[...]], axis=1)
    d = dis_ref[...]
    h = jnp.maximum(d * acc + (d * d) * xw_ref[...] + b_ref[...] + x_ref[...], 0.0)
    s_ref[...] = jnp.dot(h, wl_ref[...], preferred_element_type=jnp.float32) + bl_ref[...]


def _row_spec(w):
    return pl.BlockSpec((BN, w), lambda i: (i, 0))


def _off_spec(w, g, stride):
    return pl.BlockSpec((BN, w), lambda i, g=g, stride=stride: (g * stride + i, 0))


def _full_spec(shape):
    return pl.BlockSpec(shape, lambda i: tuple(0 for _ in shape))


def _f32(shape):
    return jax.ShapeDtypeStruct(shape, jnp.float32)


def kernel(x, edge_index, batch, n_cells, W2, b2, W3, b3, Wl, bl):
    del batch, n_cells
    src = edge_index[0]
    dst = edge_index[1]
    pad_i = jnp.full((E_PAD - E,), N1 - 1, dtype=jnp.int32)
    src_p = jnp.concatenate([src, pad_i])
    dst_p = jnp.concatenate([dst, pad_i])
    src4 = jnp.concatenate(
        [4 * src_p + g for g in range(4)]
    ).reshape(4 * (E_PAD // CHUNK), CHUNK)
    eidx = jnp.arange(E_PAD, dtype=jnp.int32)
    dump = RNG + 2 * (eidx // EPT) + (eidx // CHUNK) % 2
    dst3 = []
    for r in range(NRNG):
        base = r * RNG
        loc = jnp.where((dst_p >= base) & (dst_p < base + RNG),
                        dst_p - base, dump).astype(jnp.int32)
        dst3.append(loc.reshape(E_PAD // CHUNK, CHUNK))
    dst2d_deg = jnp.concatenate(
        [dst, jnp.full((E_PAD_D - E,), N1 - 1, dtype=jnp.int32)]
    ).reshape(E_PAD_D // CHUNK, CHUNK)
    x_p = jnp.concatenate([x, jnp.zeros((N1 - N, D), jnp.float32)])

    ones_h = jnp.ones((CHUNK, 1), jnp.float32)
    zeros1_h = jnp.zeros((ZCH, 1), jnp.float32)
    zeros16_h = jnp.zeros((ZCH, 16), jnp.float32)
    b2r = b2.reshape(1, D)
    b3r = b3.reshape(1, D)
    blr = bl.reshape(1, S2)

    degs = _deg_kernel(dst2d_deg, ones_h, zeros1_h)

    xw1 = pl.pallas_call(
        _mm_body,
        grid=(NB,),
        in_specs=[_row_spec(D), _full_spec((D, D))],
        out_specs=_row_spec(D),
        out_shape=_f32((N1, D)),
    )(x_p, W2)

    ys1, dis = pl.pallas_call(
        _scale_body,
        grid=(NB,),
        in_specs=[_row_spec(D), _off_spec(1, 0, NB), _off_spec(1, 1, NB)],
        out_specs=[_row_spec(D), _row_spec(1)],
        out_shape=[_f32((N1, D)), _f32((N1, 1))],
    )(xw1, degs, degs)

    acc1 = _spmm_kernel(ys1.reshape(4 * N1, 16), src4,
                        dst3[0], dst3[1], dst3[2], zeros16_h)

    xw2, ys2 = pl.pallas_call(
        _mid_body,
        grid=(NB,),
        in_specs=[_off_spec(16, g, GB) for g in range(4)] + [
            _row_spec(D), _row_spec(D),
            _row_spec(1), _full_spec((D, D)), _full_spec((1, D))],
        out_specs=[_row_spec(D), _row_spec(D)],
        out_shape=[_f32((N1, D)), _f32((N1, D))],
    )(acc1, acc1, acc1, acc1, xw1, x_p, dis, W3, b2r)

    acc2 = _spmm_kernel(ys2.reshape(4 * N1, 16), src4,
                        dst3[0], dst3[1], dst3[2], zeros16_h)

    s_full = pl.pallas_call(
        _head_body,
        grid=(NB,),
        in_specs=[_off_spec(16, g, GB) for g in range(4)] + [
            _row_spec(D), _row_spec(D),
            _row_spec(1), _full_spec((D, S2)), _full_spec((1, D)),
            _full_spec((1, S2))],
        out_specs=_row_spec(S2),
        out_shape=_f32((N1, S2)),
    )(acc2, acc2, acc2, acc2, xw2, x_p, dis, Wl, b3r, blr)

    return s_full[:N]
